# baseline (speedup 1.0000x reference)
import functools

import jax
import jax.numpy as jnp
from jax.experimental import pallas as pl

N_NODES = 10000
OC1 = 2


def _final_body(h_ref, w_ref, b_ref, o_ref):
    h = h_ref[...]
    logits = jnp.dot(h, w_ref[...], preferred_element_type=jnp.float32) + b_ref[...]
    m = jnp.max(logits, axis=1, keepdims=True)
    e = jnp.exp(logits - m)
    lse = jnp.log(jnp.sum(e, axis=1, keepdims=True)) + m
    o_ref[...] = logits - lse


def _final(h, fc1_W, fc1_b):
    n, d = h.shape
    dout = fc1_W.shape[1]
    blk = 1000
    return pl.pallas_call(
        _final_body,
        grid=(n // blk,),
        in_specs=[
            pl.BlockSpec((blk, d), lambda i: (i, 0)),
            pl.BlockSpec((d, dout), lambda i: (0, 0)),
            pl.BlockSpec((dout,), lambda i: (0,)),
        ],
        out_specs=pl.BlockSpec((blk, dout), lambda i: (i, 0)),
        out_shape=jax.ShapeDtypeStruct((n, dout), jnp.float32),
    )(h, fc1_W, fc1_b)


def _graph_conv(x, pos, src, dst, W_in, b_in, W_out, b_out):
    rel = pos[src] - pos[dst]
    scaling = jax.nn.relu(rel @ W_in + b_in)
    E = src.shape[0]
    label_dim = x.shape[1]
    msg = scaling.reshape(E, OC1, label_dim) * x[src][:, None, :]
    msg = msg.reshape(E, OC1 * label_dim)
    agg = jax.ops.segment_sum(msg, dst, num_segments=N_NODES)
    return agg @ W_out + b_out


def kernel(x, edge_index, pos, W_in_0, b_in_0, W_out_0, b_out_0,
           W_in_1, b_in_1, W_out_1, b_out_1, fc1_W, fc1_b):
    src = edge_index[0]
    dst = edge_index[1]
    h = _graph_conv(x, pos, src, dst, W_in_0, b_in_0, W_out_0, b_out_0)
    h = _graph_conv(h, pos, src, dst, W_in_1, b_in_1, W_out_1, b_out_1)
    return _final(h, fc1_W, fc1_b)


# speedup vs baseline: 4.0317x; 4.0317x over previous
import functools

import jax
import jax.numpy as jnp
from jax import lax
from jax.experimental import pallas as pl
from jax.experimental.pallas import tpu as pltpu
from jax.experimental.pallas import tpu_sc as plsc

N = 10000
E = 160000
F = 128
NS = 16
CHUNK = 80
EPT = E // NS
NCHUNK = EPT // CHUNK
ZCH = 40
NZCH = N // ZCH
IPAD = 8


def _edge_body(feats, pos3, srcr, dstr, wr, br, out,
               acc, posv, sidx, didx, rowsv, relv, wv, bv, obuf, sem, sem2):
    c = lax.axis_index("c")
    s = lax.axis_index("s")

    pltpu.sync_copy(pos3, posv)
    pltpu.sync_copy(wr.at[c], wv)
    pltpu.sync_copy(br.at[c], bv)
    pltpu.sync_copy(srcr.at[s, 0], sidx.at[0])
    pltpu.sync_copy(dstr.at[s, 0], didx.at[0])

    zeros16 = jnp.zeros((16,), jnp.float32)
    iota16 = lax.iota(jnp.int32, 16)

    def zrow(i, carry):
        for f in range(8):
            obuf[i, pl.ds(16 * f, 16)] = zeros16
        return carry

    lax.fori_loop(0, ZCH, zrow, 0)
    for t in range(16):
        cid = s + NS * t

        @pl.when(cid < NZCH)
        def _():
            pltpu.sync_copy(obuf, acc.at[pl.ds(cid * ZCH, ZCH)])

    plsc.subcore_barrier()

    wregs = [[wv[k, pl.ds(16 * f, 16)] for f in range(8)] for k in range(3)]
    bregs = [bv[0, pl.ds(16 * f, 16)] for f in range(8)]

    def chunk_body(j, carry):
        b = lax.rem(j, 2)
        nb = 1 - b
        jnext = jnp.minimum(j + 1, NCHUNK - 1)
        cps = pltpu.async_copy(srcr.at[s, jnext], sidx.at[nb], sem2)
        cpd = pltpu.async_copy(dstr.at[s, jnext], didx.at[nb], sem2)
        srcj = sidx.at[b, 0]
        dstj = didx.at[b, 0]
        pltpu.async_copy(feats.at[srcj], rowsv, sem).wait()
        for g in range(5):
            s16 = sidx[b, 0, pl.ds(16 * g, 16)]
            d16 = didx[b, 0, pl.ds(16 * g, 16)]
            for k in range(3):
                r = (plsc.load_gather(posv, [s16 + (k * N)])
                     - plsc.load_gather(posv, [d16 + (k * N)]))
                relv[pl.ds(k * CHUNK + 16 * g, 16)] = r

        def edge_body(e, ecarry):
            rx = plsc.load_gather(relv, [jnp.full((16,), 0, jnp.int32) + e])
            ry = plsc.load_gather(relv, [jnp.full((16,), CHUNK, jnp.int32) + e])
            rz = plsc.load_gather(relv, [jnp.full((16,), 2 * CHUNK, jnp.int32) + e])
            for f in range(8):
                z = rx * wregs[0][f] + ry * wregs[1][f] + rz * wregs[2][f] + bregs[f]
                sc = jnp.maximum(z, 0.0)
                rowsv[e, pl.ds(16 * f, 16)] = sc * rowsv[e, pl.ds(16 * f, 16)]
            return ecarry

        lax.fori_loop(0, CHUNK, edge_body, 0)
        pltpu.sync_copy(rowsv, acc.at[dstj], add=True)
        cps.wait()
        cpd.wait()
        return carry

    lax.fori_loop(0, NCHUNK, chunk_body, 0)
    plsc.subcore_barrier()

    for t in range(16):
        cid = s + NS * t

        @pl.when(cid < NZCH)
        def _():
            pltpu.sync_copy(acc.at[pl.ds(cid * ZCH, ZCH)], obuf)
            pltpu.sync_copy(obuf, out.at[c, pl.ds(cid * ZCH, ZCH)])


_edge_agg = pl.kernel(
    _edge_body,
    out_type=jax.ShapeDtypeStruct((2, N, F), jnp.float32),
    mesh=plsc.VectorSubcoreMesh(core_axis_name="c", subcore_axis_name="s"),
    compiler_params=pltpu.CompilerParams(needs_layout_passes=False),
    scratch_types=[
        pltpu.VMEM_SHARED((N, F), jnp.float32),
        pltpu.VMEM((3 * N,), jnp.float32),
        pltpu.VMEM((2, IPAD, CHUNK), jnp.int32),
        pltpu.VMEM((2, IPAD, CHUNK), jnp.int32),
        pltpu.VMEM((CHUNK, F), jnp.float32),
        pltpu.VMEM((3 * CHUNK,), jnp.float32),
        pltpu.VMEM((3, F), jnp.float32),
        pltpu.VMEM((1, F), jnp.float32),
        pltpu.VMEM((ZCH, F), jnp.float32),
        pltpu.SemaphoreType.DMA,
        pltpu.SemaphoreType.DMA,
    ],
)


def _dense_mid_body(a_ref, w_ref, b_ref, o_ref):
    o = jnp.dot(a_ref[0], w_ref[0], preferred_element_type=jnp.float32)
    o += jnp.dot(a_ref[1], w_ref[1], preferred_element_type=jnp.float32)
    o_ref[...] = o + b_ref[...]


def _dense_mid(agg, w_r, b):
    blk = 1000
    return pl.pallas_call(
        _dense_mid_body,
        grid=(N // blk,),
        in_specs=[
            pl.BlockSpec((2, blk, F), lambda i: (0, i, 0)),
            pl.BlockSpec((2, F, F), lambda i: (0, 0, 0)),
            pl.BlockSpec((F,), lambda i: (0,)),
        ],
        out_specs=pl.BlockSpec((blk, F), lambda i: (i, 0)),
        out_shape=jax.ShapeDtypeStruct((N, F), jnp.float32),
    )(agg, w_r, b)


def _dense_final_body(a_ref, w_ref, b_ref, fw_ref, fb_ref, o_ref):
    h = jnp.dot(a_ref[0], w_ref[0], preferred_element_type=jnp.float32)
    h += jnp.dot(a_ref[1], w_ref[1], preferred_element_type=jnp.float32)
    h += b_ref[...]
    logits = jnp.dot(h, fw_ref[...], preferred_element_type=jnp.float32) + fb_ref[...]
    m = jnp.max(logits, axis=1, keepdims=True)
    lse = jnp.log(jnp.sum(jnp.exp(logits - m), axis=1, keepdims=True)) + m
    o_ref[...] = logits - lse


def _dense_final(agg, w_r, b, fc1_W, fc1_b):
    blk = 1000
    dout = fc1_W.shape[1]
    return pl.pallas_call(
        _dense_final_body,
        grid=(N // blk,),
        in_specs=[
            pl.BlockSpec((2, blk, F), lambda i: (0, i, 0)),
            pl.BlockSpec((2, F, F), lambda i: (0, 0, 0)),
            pl.BlockSpec((F,), lambda i: (0,)),
            pl.BlockSpec((F, dout), lambda i: (0, 0)),
            pl.BlockSpec((dout,), lambda i: (0,)),
        ],
        out_specs=pl.BlockSpec((blk, dout), lambda i: (i, 0)),
        out_shape=jax.ShapeDtypeStruct((N, dout), jnp.float32),
    )(agg, w_r, b, fc1_W, fc1_b)


def kernel(x, edge_index, pos, W_in_0, b_in_0, W_out_0, b_out_0,
           W_in_1, b_in_1, W_out_1, b_out_1, fc1_W, fc1_b):
    src_r = jnp.broadcast_to(
        edge_index[0].reshape(NS, NCHUNK, 1, CHUNK), (NS, NCHUNK, IPAD, CHUNK))
    dst_r = jnp.broadcast_to(
        edge_index[1].reshape(NS, NCHUNK, 1, CHUNK), (NS, NCHUNK, IPAD, CHUNK))
    pos3 = pos.T.reshape(-1)

    w0_r = W_in_0.reshape(3, 2, F).transpose(1, 0, 2)
    b0_r = b_in_0.reshape(2, 1, F)
    w1_r = W_in_1.reshape(3, 2, F).transpose(1, 0, 2)
    b1_r = b_in_1.reshape(2, 1, F)
    wo0_r = W_out_0.reshape(2, F, F)
    wo1_r = W_out_1.reshape(2, F, F)

    agg0 = _edge_agg(x, pos3, src_r, dst_r, w0_r, b0_r)
    h = _dense_mid(agg0, wo0_r, b_out_0)
    agg1 = _edge_agg(h, pos3, src_r, dst_r, w1_r, b1_r)
    return _dense_final(agg1, wo1_r, b_out_1, fc1_W, fc1_b)
